# baseline (speedup 1.0000x reference)
import jax
import jax.numpy as jnp
from jax import lax
from jax.experimental import pallas as pl
from jax.experimental.pallas import tpu as pltpu
from jax.experimental.pallas import tpu_sc as plsc

_B, _C, _H, _W = 8, 96, 224, 224
_HW = _H * _W
_CHUNK = 6272
_CPAD = 128


def _sc_gather_body(gamma_hbm, beta_hbm, labels_hbm,
                    g_out, b_out, idx_v, rows_v, sem):
    wid = lax.axis_index("s") * 2 + lax.axis_index("c")

    @pl.when(wid == 0)
    def _():
        pltpu.sync_copy(labels_hbm, idx_v)
        pltpu.async_copy(gamma_hbm.at[idx_v], rows_v, sem).wait()
        pltpu.sync_copy(rows_v, g_out)

    @pl.when(wid == 1)
    def _():
        pltpu.sync_copy(labels_hbm, idx_v)
        pltpu.async_copy(beta_hbm.at[idx_v], rows_v, sem).wait()
        pltpu.sync_copy(rows_v, b_out)


_SC_GATHER_CACHE = []


def _sc_gather(gamma_table, beta_table, labels):
    if not _SC_GATHER_CACHE:
        _SC_GATHER_CACHE.append(pl.kernel(
            _sc_gather_body,
            out_type=(jax.ShapeDtypeStruct((_B, _CPAD), jnp.float32),
                      jax.ShapeDtypeStruct((_B, _CPAD), jnp.float32)),
            mesh=plsc.VectorSubcoreMesh(core_axis_name="c",
                                        subcore_axis_name="s"),
            scratch_types=[
                pltpu.VMEM((_B,), jnp.int32),
                pltpu.VMEM((_B, _CPAD), jnp.float32),
                pltpu.SemaphoreType.DMA,
            ],
        ))
    return _SC_GATHER_CACHE[0](gamma_table, beta_table, labels)


def _film_body(g_ref, b_ref, x_ref, o_ref):
    o_ref[...] = x_ref[...] * g_ref[...] + b_ref[...]


def _film_tc(x3, g3, b3):
    return pl.pallas_call(
        _film_body,
        grid=(_B, _HW // _CHUNK),
        in_specs=[
            pl.BlockSpec((1, _C, 1), lambda i, j: (i, 0, 0)),
            pl.BlockSpec((1, _C, 1), lambda i, j: (i, 0, 0)),
            pl.BlockSpec((1, _C, _CHUNK), lambda i, j: (i, 0, j)),
        ],
        out_specs=pl.BlockSpec((1, _C, _CHUNK), lambda i, j: (i, 0, j)),
        out_shape=jax.ShapeDtypeStruct((_B, _C, _HW), jnp.float32),
    )(g3, b3, x3)


def kernel(x, labels, gamma_table, beta_table):
    labels = labels.astype(jnp.int32)
    pad = ((0, 0), (0, _CPAD - _C))
    g_rows, b_rows = _sc_gather(jnp.pad(gamma_table, pad),
                                jnp.pad(beta_table, pad), labels)
    g_rows, b_rows = g_rows[:, :_C], b_rows[:, :_C]
    out3 = _film_tc(x.reshape(_B, _C, _HW),
                    g_rows.reshape(_B, _C, 1), b_rows.reshape(_B, _C, 1))
    return out3.reshape(_B, _C, _H, _W)
